# baseline (speedup 1.0000x reference)
import functools

import jax
import jax.numpy as jnp
from jax import lax
from jax.experimental import pallas as pl
from jax.experimental.pallas import tpu as pltpu
from jax.experimental.pallas import tpu_sc as plsc

N = 10000
E = 320000
L = 16
NCORES = 2
NSUB = 16
NWORK = NCORES * NSUB
EPW = E // NWORK
EC = 8000

NB = 2000


def _mesh():
    return plsc.VectorSubcoreMesh(
        core_axis_name="c", subcore_axis_name="s",
        num_cores=NCORES, num_subcores=NSUB)


def _wid():
    return lax.axis_index("s") * NCORES + lax.axis_index("c")


def _deg_body(ew_hbm, src_hbm, degp_out, acc_v, src_v, ew_v):
    wid = _wid()
    base = wid * EPW
    pltpu.sync_copy(src_hbm.at[pl.ds(base, EPW)], src_v)
    pltpu.sync_copy(ew_hbm.at[pl.ds(base, EPW)], ew_v)

    @plsc.parallel_loop(0, N // L, unroll=8)
    def zero(i):
        acc_v[pl.ds(i * L, L)] = jnp.zeros((L,), jnp.float32)

    @plsc.parallel_loop(0, EPW // L, unroll=8)
    def body(g):
        s = src_v[pl.ds(g * L, L)]
        w = ew_v[pl.ds(g * L, L)]
        plsc.addupdate_scatter(acc_v, [s], w)
    pltpu.sync_copy(acc_v, degp_out.at[wid])


def _nw_body(dinv_hbm, src_hbm, dst_hbm, ew_hbm, nw_out, pk_out,
             dinv_v, src_v, dst_v, ew_v, nw_v, pk_v):
    wid = _wid()
    base = wid * EPW
    pltpu.sync_copy(dinv_hbm, dinv_v)
    pltpu.sync_copy(src_hbm.at[pl.ds(base, EPW)], src_v)
    pltpu.sync_copy(dst_hbm.at[pl.ds(base, EPW)], dst_v)
    pltpu.sync_copy(ew_hbm.at[pl.ds(base, EPW)], ew_v)

    @plsc.parallel_loop(0, EPW // L, unroll=8)
    def body(g):
        s = src_v[pl.ds(g * L, L)]
        d = dst_v[pl.ds(g * L, L)]
        w = ew_v[pl.ds(g * L, L)]
        a = plsc.load_gather(dinv_v, [s])
        b = plsc.load_gather(dinv_v, [d])
        nw_v[pl.ds(g * L, L)] = a * w * b
        pk_v[pl.ds(g * L, L)] = s | (d << 16)
    pltpu.sync_copy(nw_v, nw_out.at[pl.ds(base, EPW)])
    pltpu.sync_copy(pk_v, pk_out.at[pl.ds(base, EPW)])


def _make_prop(F, fpw):
    assert F == fpw * NSUB
    epw = E // 2
    ec = EC
    nchunks = epw // ec
    assert nchunks % 2 == 0 and epw % ec == 0

    def body(pt_hbm, pk_hbm, nw_hbm, gt_out, *scratch):
        pf = scratch[:fpw]
        gf = scratch[fpw:2 * fpw]
        pk_b = scratch[2 * fpw:2 * fpw + 2]
        nw_b = scratch[2 * fpw + 2:2 * fpw + 4]
        sems = scratch[2 * fpw + 4]
        ehalf = lax.axis_index("c")
        f0 = lax.axis_index("s") * fpw
        for j in range(fpw):
            pltpu.sync_copy(pt_hbm.at[f0 + j], pf[j])

        def edge_copies(c, b):
            base = ehalf * epw + c * ec
            return (
                pltpu.make_async_copy(
                    pk_hbm.at[pl.ds(base, ec)], pk_b[b], sems.at[2 * b]),
                pltpu.make_async_copy(
                    nw_hbm.at[pl.ds(base, ec)], nw_b[b], sems.at[2 * b + 1]),
            )

        def issue(c, b):
            for cp in edge_copies(c, b):
                cp.start()

        issue(0, 0)
        issue(1, 1)

        @plsc.parallel_loop(0, N // L, unroll=8)
        def zero(i):
            z = jnp.zeros((L,), jnp.float32)
            for j in range(fpw):
                gf[j][pl.ds(i * L, L)] = z

        def process(c, b):
            for cp in edge_copies(c, b):
                cp.wait()

            @plsc.parallel_loop(0, ec // L, unroll=16)
            def grp(g):
                pk = pk_b[b][pl.ds(g * L, L)]
                w = nw_b[b][pl.ds(g * L, L)]
                s = pk & 0xFFFF
                d = lax.shift_right_logical(pk, 16)
                for j in range(fpw):
                    v = plsc.load_gather(pf[j], [s])
                    plsc.addupdate_scatter(gf[j], [d], v * w)

        def chunk2(k, carry):
            c0 = 2 * k
            process(c0, 0)

            @pl.when(c0 + 2 < nchunks)
            def _():
                issue(c0 + 2, 0)
            process(c0 + 1, 1)

            @pl.when(c0 + 3 < nchunks)
            def _():
                issue(c0 + 3, 1)
            return carry
        lax.fori_loop(0, nchunks // 2, chunk2, 0)

        for j in range(fpw):
            pltpu.sync_copy(gf[j], gt_out.at[ehalf, f0 + j])

    scratch = ([pltpu.VMEM((N,), jnp.float32)] * (2 * fpw)
               + [pltpu.VMEM((ec,), jnp.int32)] * 2
               + [pltpu.VMEM((ec,), jnp.float32)] * 2
               + [pltpu.SemaphoreType.DMA((4,))])
    return pl.kernel(
        body,
        out_type=jax.ShapeDtypeStruct((2, F, N), jnp.float32),
        mesh=_mesh(),
        scratch_types=scratch,
        compiler_params=pltpu.CompilerParams(needs_layout_passes=False),
    )


def _deg_kernel():
    return pl.kernel(
        _deg_body,
        out_type=jax.ShapeDtypeStruct((NWORK, N), jnp.float32),
        mesh=_mesh(),
        scratch_types=[pltpu.VMEM((N,), jnp.float32),
                       pltpu.VMEM((EPW,), jnp.int32),
                       pltpu.VMEM((EPW,), jnp.float32)],
        compiler_params=pltpu.CompilerParams(needs_layout_passes=False),
    )


def _nw_kernel():
    return pl.kernel(
        _nw_body,
        out_type=[jax.ShapeDtypeStruct((E,), jnp.float32),
                  jax.ShapeDtypeStruct((E,), jnp.int32)],
        mesh=_mesh(),
        scratch_types=[pltpu.VMEM((N,), jnp.float32),
                       pltpu.VMEM((EPW,), jnp.int32),
                       pltpu.VMEM((EPW,), jnp.int32),
                       pltpu.VMEM((EPW,), jnp.float32),
                       pltpu.VMEM((EPW,), jnp.float32),
                       pltpu.VMEM((EPW,), jnp.int32)],
        compiler_params=pltpu.CompilerParams(needs_layout_passes=False),
    )


def _tc1_body(xt_ref, degp_ref, wd_ref, wp_ref, d1_ref, p1_ref, dinv_ref):
    deg = jnp.sum(degp_ref[...], axis=0, keepdims=True)
    deg_safe = jnp.where(deg > 0, deg, 1.0)
    dinv_ref[...] = jnp.where(deg > 0, lax.rsqrt(deg_safe), 0.0)
    xt = xt_ref[...]
    d1_ref[...] = jnp.dot(wd_ref[...], xt, preferred_element_type=jnp.float32)
    p1_ref[...] = jnp.dot(wp_ref[...], xt, preferred_element_type=jnp.float32)


def _tc1_call(xt, degp, wdt, wpt):
    return pl.pallas_call(
        _tc1_body,
        out_shape=[
            jax.ShapeDtypeStruct((64, N), jnp.float32),
            jax.ShapeDtypeStruct((64, N), jnp.float32),
            jax.ShapeDtypeStruct((1, N), jnp.float32),
        ],
    )(xt, degp, wdt, wpt)


def _tc2_body(d1_ref, g1_ref, bc_ref, wd_ref, wp_ref, d2_ref, p2_ref):
    pre = d1_ref[...] - (g1_ref[0] + g1_ref[1]) + bc_ref[...]
    z = jax.nn.sigmoid(pre[:32, :])
    ht = jnp.tanh(pre[32:, :])
    h = jax.nn.relu((1.0 - z) * ht)
    d2_ref[...] = jnp.dot(wd_ref[...], h, preferred_element_type=jnp.float32)
    p2_ref[...] = jnp.dot(wp_ref[...], h, preferred_element_type=jnp.float32)


def _tc2_call(d1t, g1t, bct, wdt, wpt):
    return pl.pallas_call(
        _tc2_body,
        out_shape=[
            jax.ShapeDtypeStruct((32, N), jnp.float32),
            jax.ShapeDtypeStruct((32, N), jnp.float32),
        ],
    )(d1t, g1t, bct, wdt, wpt)


def _tc3_body(d2_ref, g2_ref, bc_ref, lw_ref, lb_ref, out_ref):
    pre = d2_ref[...] - (g2_ref[0] + g2_ref[1]) + bc_ref[...]
    z = jax.nn.sigmoid(pre[:16, :])
    ht = jnp.tanh(pre[16:, :])
    h = jax.nn.relu((1.0 - z) * ht)
    logits = jnp.dot(lw_ref[...], h, preferred_element_type=jnp.float32)
    logits = logits + lb_ref[...]
    m = jnp.max(logits, axis=0, keepdims=True)
    shifted = logits - m
    ssum = jnp.sum(jnp.exp(shifted), axis=0, keepdims=True)
    out_ref[...] = shifted - jnp.log(ssum)


def _tc3_call(d2t, g2t, bct, lwt, lbt):
    return pl.pallas_call(
        _tc3_body,
        out_shape=jax.ShapeDtypeStruct((10, N), jnp.float32),
    )(d2t, g2t, bct, lwt, lbt)


def kernel(x, edge_index, edge_weight, params):
    src = edge_index[0]
    dst = edge_index[1]
    g1, g2 = params["gru1"], params["gru2"]
    wd1t = jnp.concatenate([g1["xz"]["W"][0], g1["xh"]["W"][0]], axis=1).T
    wp1t = jnp.concatenate([g1["xz"]["W"][1], g1["xh"]["W"][1]], axis=1).T
    bc1t = jnp.concatenate([g1["xz"]["b"] + g1["hz"]["b"],
                            g1["xh"]["b"] + g1["hh"]["b"]])[:, None]
    wd2t = jnp.concatenate([g2["xz"]["W"][0], g2["xh"]["W"][0]], axis=1).T
    wp2t = jnp.concatenate([g2["xz"]["W"][1], g2["xh"]["W"][1]], axis=1).T
    bc2t = jnp.concatenate([g2["xz"]["b"] + g2["hz"]["b"],
                            g2["xh"]["b"] + g2["hh"]["b"]])[:, None]
    lwt = params["lin_W"].T
    lbt = params["lin_b"][:, None]

    degp = _deg_kernel()(edge_weight, src)
    d1t, p1t, dinv2d = _tc1_call(x.T, degp, wd1t, wp1t)
    dinv = dinv2d.reshape(N)
    nw, pk = _nw_kernel()(dinv, src, dst, edge_weight)
    g1t = _make_prop(64, 4)(p1t, pk, nw)
    d2t, p2t = _tc2_call(d1t, g1t, bc1t, wd2t, wp2t)
    g2t = _make_prop(32, 2)(p2t, pk, nw)
    return _tc3_call(d2t, g2t, bc2t, lwt, lbt).T


# speedup vs baseline: 1.0138x; 1.0138x over previous
import functools

import jax
import jax.numpy as jnp
from jax import lax
from jax.experimental import pallas as pl
from jax.experimental.pallas import tpu as pltpu
from jax.experimental.pallas import tpu_sc as plsc

N = 10000
E = 320000
L = 16
NCORES = 2
NSUB = 16
NWORK = NCORES * NSUB
EPW = E // NWORK
EC = 8000

NB = 2000


def _mesh():
    return plsc.VectorSubcoreMesh(
        core_axis_name="c", subcore_axis_name="s",
        num_cores=NCORES, num_subcores=NSUB)


def _wid():
    return lax.axis_index("s") * NCORES + lax.axis_index("c")


def _deg_body(ew_hbm, src_hbm, degp_out, acc_v, src_v, ew_v):
    wid = _wid()
    base = wid * EPW
    pltpu.sync_copy(src_hbm.at[pl.ds(base, EPW)], src_v)
    pltpu.sync_copy(ew_hbm.at[pl.ds(base, EPW)], ew_v)

    @plsc.parallel_loop(0, N // L, unroll=8)
    def zero(i):
        acc_v[pl.ds(i * L, L)] = jnp.zeros((L,), jnp.float32)

    @plsc.parallel_loop(0, EPW // L, unroll=8)
    def body(g):
        s = src_v[pl.ds(g * L, L)]
        w = ew_v[pl.ds(g * L, L)]
        plsc.addupdate_scatter(acc_v, [s], w)
    pltpu.sync_copy(acc_v, degp_out.at[wid])


def _nw_body(dinv_hbm, src_hbm, dst_hbm, ew_hbm, nw_out, pk_out,
             dinv_v, src_v, dst_v, ew_v, nw_v, pk_v):
    wid = _wid()
    base = wid * EPW
    pltpu.sync_copy(dinv_hbm, dinv_v)
    pltpu.sync_copy(src_hbm.at[pl.ds(base, EPW)], src_v)
    pltpu.sync_copy(dst_hbm.at[pl.ds(base, EPW)], dst_v)
    pltpu.sync_copy(ew_hbm.at[pl.ds(base, EPW)], ew_v)

    @plsc.parallel_loop(0, EPW // L, unroll=8)
    def body(g):
        s = src_v[pl.ds(g * L, L)]
        d = dst_v[pl.ds(g * L, L)]
        w = ew_v[pl.ds(g * L, L)]
        a = plsc.load_gather(dinv_v, [s])
        b = plsc.load_gather(dinv_v, [d])
        nw_v[pl.ds(g * L, L)] = a * w * b
        pk_v[pl.ds(g * L, L)] = s | (d << 16)
    pltpu.sync_copy(nw_v, nw_out.at[pl.ds(base, EPW)])
    pltpu.sync_copy(pk_v, pk_out.at[pl.ds(base, EPW)])


def _make_prop(F, fpw):
    assert F == fpw * NSUB
    epw = E // 2
    ec = EC
    nchunks = epw // ec
    assert nchunks % 2 == 0 and epw % ec == 0

    def body(pt_hbm, pk_hbm, nw_hbm, gt_out, *scratch):
        pf = scratch[:fpw]
        gf = scratch[fpw:2 * fpw]
        pk_b = scratch[2 * fpw:2 * fpw + 2]
        nw_b = scratch[2 * fpw + 2:2 * fpw + 4]
        sems = scratch[2 * fpw + 4]
        ehalf = lax.axis_index("c")
        f0 = lax.axis_index("s") * fpw
        for j in range(fpw):
            pltpu.sync_copy(pt_hbm.at[f0 + j], pf[j])

        def edge_copies(c, b):
            base = ehalf * epw + c * ec
            return (
                pltpu.make_async_copy(
                    pk_hbm.at[pl.ds(base, ec)], pk_b[b], sems.at[2 * b]),
                pltpu.make_async_copy(
                    nw_hbm.at[pl.ds(base, ec)], nw_b[b], sems.at[2 * b + 1]),
            )

        def issue(c, b):
            for cp in edge_copies(c, b):
                cp.start()

        issue(0, 0)
        issue(1, 1)

        @plsc.parallel_loop(0, N // L, unroll=8)
        def zero(i):
            z = jnp.zeros((L,), jnp.float32)
            for j in range(fpw):
                gf[j][pl.ds(i * L, L)] = z

        def process(c, b):
            for cp in edge_copies(c, b):
                cp.wait()

            @plsc.parallel_loop(0, ec // L, unroll=8)
            def grp(g):
                pk = pk_b[b][pl.ds(g * L, L)]
                w = nw_b[b][pl.ds(g * L, L)]
                s = pk & 0xFFFF
                d = lax.shift_right_logical(pk, 16)
                for j in range(fpw):
                    v = plsc.load_gather(pf[j], [s])
                    plsc.addupdate_scatter(gf[j], [d], v * w)

        def chunk2(k, carry):
            c0 = 2 * k
            process(c0, 0)

            @pl.when(c0 + 2 < nchunks)
            def _():
                issue(c0 + 2, 0)
            process(c0 + 1, 1)

            @pl.when(c0 + 3 < nchunks)
            def _():
                issue(c0 + 3, 1)
            return carry
        lax.fori_loop(0, nchunks // 2, chunk2, 0)

        for j in range(fpw):
            pltpu.sync_copy(gf[j], gt_out.at[ehalf, f0 + j])

    scratch = ([pltpu.VMEM((N,), jnp.float32)] * (2 * fpw)
               + [pltpu.VMEM((ec,), jnp.int32)] * 2
               + [pltpu.VMEM((ec,), jnp.float32)] * 2
               + [pltpu.SemaphoreType.DMA((4,))])
    return pl.kernel(
        body,
        out_type=jax.ShapeDtypeStruct((2, F, N), jnp.float32),
        mesh=_mesh(),
        scratch_types=scratch,
        compiler_params=pltpu.CompilerParams(needs_layout_passes=False),
    )


def _deg_kernel():
    return pl.kernel(
        _deg_body,
        out_type=jax.ShapeDtypeStruct((NWORK, N), jnp.float32),
        mesh=_mesh(),
        scratch_types=[pltpu.VMEM((N,), jnp.float32),
                       pltpu.VMEM((EPW,), jnp.int32),
                       pltpu.VMEM((EPW,), jnp.float32)],
        compiler_params=pltpu.CompilerParams(needs_layout_passes=False),
    )


def _nw_kernel():
    return pl.kernel(
        _nw_body,
        out_type=[jax.ShapeDtypeStruct((E,), jnp.float32),
                  jax.ShapeDtypeStruct((E,), jnp.int32)],
        mesh=_mesh(),
        scratch_types=[pltpu.VMEM((N,), jnp.float32),
                       pltpu.VMEM((EPW,), jnp.int32),
                       pltpu.VMEM((EPW,), jnp.int32),
                       pltpu.VMEM((EPW,), jnp.float32),
                       pltpu.VMEM((EPW,), jnp.float32),
                       pltpu.VMEM((EPW,), jnp.int32)],
        compiler_params=pltpu.CompilerParams(needs_layout_passes=False),
    )


def _tc1_body(xt_ref, degp_ref, wd_ref, wp_ref, d1_ref, p1_ref, dinv_ref):
    deg = jnp.sum(degp_ref[...], axis=0, keepdims=True)
    deg_safe = jnp.where(deg > 0, deg, 1.0)
    dinv_ref[...] = jnp.where(deg > 0, lax.rsqrt(deg_safe), 0.0)
    xt = xt_ref[...]
    d1_ref[...] = jnp.dot(wd_ref[...], xt, preferred_element_type=jnp.float32)
    p1_ref[...] = jnp.dot(wp_ref[...], xt, preferred_element_type=jnp.float32)


def _tc1_call(xt, degp, wdt, wpt):
    return pl.pallas_call(
        _tc1_body,
        out_shape=[
            jax.ShapeDtypeStruct((64, N), jnp.float32),
            jax.ShapeDtypeStruct((64, N), jnp.float32),
            jax.ShapeDtypeStruct((1, N), jnp.float32),
        ],
    )(xt, degp, wdt, wpt)


def _tc2_body(d1_ref, g1_ref, bc_ref, wd_ref, wp_ref, d2_ref, p2_ref):
    pre = d1_ref[...] - (g1_ref[0] + g1_ref[1]) + bc_ref[...]
    z = jax.nn.sigmoid(pre[:32, :])
    ht = jnp.tanh(pre[32:, :])
    h = jax.nn.relu((1.0 - z) * ht)
    d2_ref[...] = jnp.dot(wd_ref[...], h, preferred_element_type=jnp.float32)
    p2_ref[...] = jnp.dot(wp_ref[...], h, preferred_element_type=jnp.float32)


def _tc2_call(d1t, g1t, bct, wdt, wpt):
    return pl.pallas_call(
        _tc2_body,
        out_shape=[
            jax.ShapeDtypeStruct((32, N), jnp.float32),
            jax.ShapeDtypeStruct((32, N), jnp.float32),
        ],
    )(d1t, g1t, bct, wdt, wpt)


def _tc3_body(d2_ref, g2_ref, bc_ref, lw_ref, lb_ref, out_ref):
    pre = d2_ref[...] - (g2_ref[0] + g2_ref[1]) + bc_ref[...]
    z = jax.nn.sigmoid(pre[:16, :])
    ht = jnp.tanh(pre[16:, :])
    h = jax.nn.relu((1.0 - z) * ht)
    logits = jnp.dot(lw_ref[...], h, preferred_element_type=jnp.float32)
    logits = logits + lb_ref[...]
    m = jnp.max(logits, axis=0, keepdims=True)
    shifted = logits - m
    ssum = jnp.sum(jnp.exp(shifted), axis=0, keepdims=True)
    out_ref[...] = shifted - jnp.log(ssum)


def _tc3_call(d2t, g2t, bct, lwt, lbt):
    return pl.pallas_call(
        _tc3_body,
        out_shape=jax.ShapeDtypeStruct((10, N), jnp.float32),
    )(d2t, g2t, bct, lwt, lbt)


def kernel(x, edge_index, edge_weight, params):
    src = edge_index[0]
    dst = edge_index[1]
    g1, g2 = params["gru1"], params["gru2"]
    wd1t = jnp.concatenate([g1["xz"]["W"][0], g1["xh"]["W"][0]], axis=1).T
    wp1t = jnp.concatenate([g1["xz"]["W"][1], g1["xh"]["W"][1]], axis=1).T
    bc1t = jnp.concatenate([g1["xz"]["b"] + g1["hz"]["b"],
                            g1["xh"]["b"] + g1["hh"]["b"]])[:, None]
    wd2t = jnp.concatenate([g2["xz"]["W"][0], g2["xh"]["W"][0]], axis=1).T
    wp2t = jnp.concatenate([g2["xz"]["W"][1], g2["xh"]["W"][1]], axis=1).T
    bc2t = jnp.concatenate([g2["xz"]["b"] + g2["hz"]["b"],
                            g2["xh"]["b"] + g2["hh"]["b"]])[:, None]
    lwt = params["lin_W"].T
    lbt = params["lin_b"][:, None]

    degp = _deg_kernel()(edge_weight, src)
    d1t, p1t, dinv2d = _tc1_call(x.T, degp, wd1t, wp1t)
    dinv = dinv2d.reshape(N)
    nw, pk = _nw_kernel()(dinv, src, dst, edge_weight)
    g1t = _make_prop(64, 4)(p1t, pk, nw)
    d2t, p2t = _tc2_call(d1t, g1t, bc1t, wd2t, wp2t)
    g2t = _make_prop(32, 2)(p2t, pk, nw)
    return _tc3_call(d2t, g2t, bc2t, lwt, lbt).T


# speedup vs baseline: 1.0396x; 1.0254x over previous
import functools

import jax
import jax.numpy as jnp
from jax import lax
from jax.experimental import pallas as pl
from jax.experimental.pallas import tpu as pltpu
from jax.experimental.pallas import tpu_sc as plsc

N = 10000
E = 320000
L = 16
NCORES = 2
NSUB = 16
NWORK = NCORES * NSUB
EPW = E // NWORK
EC = 8000

NB = 2000


def _mesh():
    return plsc.VectorSubcoreMesh(
        core_axis_name="c", subcore_axis_name="s",
        num_cores=NCORES, num_subcores=NSUB)


def _wid():
    return lax.axis_index("s") * NCORES + lax.axis_index("c")


def _deg_body(ew_hbm, src_hbm, degp_out, acc_v, src_v, ew_v, sems):
    wid = _wid()
    base = wid * EPW
    cps = (pltpu.make_async_copy(src_hbm.at[pl.ds(base, EPW)], src_v,
                                 sems.at[0]),
           pltpu.make_async_copy(ew_hbm.at[pl.ds(base, EPW)], ew_v,
                                 sems.at[1]))
    for cp in cps:
        cp.start()

    @plsc.parallel_loop(0, N // L, unroll=8)
    def zero(i):
        acc_v[pl.ds(i * L, L)] = jnp.zeros((L,), jnp.float32)
    for cp in cps:
        cp.wait()

    @plsc.parallel_loop(0, EPW // L, unroll=8)
    def body(g):
        s = src_v[pl.ds(g * L, L)]
        w = ew_v[pl.ds(g * L, L)]
        plsc.addupdate_scatter(acc_v, [s], w)
    pltpu.sync_copy(acc_v, degp_out.at[wid])


def _nw_body(dinv_hbm, src_hbm, dst_hbm, ew_hbm, nw_out, pk_out,
             dinv_v, src_v, dst_v, ew_v, nw_v, pk_v, sems):
    wid = _wid()
    base = wid * EPW
    cps = (pltpu.make_async_copy(dinv_hbm, dinv_v, sems.at[0]),
           pltpu.make_async_copy(src_hbm.at[pl.ds(base, EPW)], src_v,
                                 sems.at[1]),
           pltpu.make_async_copy(dst_hbm.at[pl.ds(base, EPW)], dst_v,
                                 sems.at[2]),
           pltpu.make_async_copy(ew_hbm.at[pl.ds(base, EPW)], ew_v,
                                 sems.at[3]))
    for cp in cps:
        cp.start()
    for cp in cps:
        cp.wait()

    @plsc.parallel_loop(0, EPW // L, unroll=8)
    def body(g):
        s = src_v[pl.ds(g * L, L)]
        d = dst_v[pl.ds(g * L, L)]
        w = ew_v[pl.ds(g * L, L)]
        a = plsc.load_gather(dinv_v, [s])
        b = plsc.load_gather(dinv_v, [d])
        nw_v[pl.ds(g * L, L)] = a * w * b
        pk_v[pl.ds(g * L, L)] = s | (d << 16)
    pltpu.sync_copy(nw_v, nw_out.at[pl.ds(base, EPW)])
    pltpu.sync_copy(pk_v, pk_out.at[pl.ds(base, EPW)])


def _make_prop(F, fpw):
    assert F == fpw * NSUB
    epw = E // 2
    ec = EC
    nchunks = epw // ec
    assert nchunks % 2 == 0 and epw % ec == 0

    def body(pt_hbm, pk_hbm, nw_hbm, gt_out, *scratch):
        pf = scratch[:fpw]
        gf = scratch[fpw:2 * fpw]
        pk_b = scratch[2 * fpw:2 * fpw + 2]
        nw_b = scratch[2 * fpw + 2:2 * fpw + 4]
        sems = scratch[2 * fpw + 4]
        psems = scratch[2 * fpw + 5]
        ehalf = lax.axis_index("c")
        f0 = lax.axis_index("s") * fpw
        pf_cps = [pltpu.make_async_copy(pt_hbm.at[f0 + j], pf[j],
                                        psems.at[j])
                  for j in range(fpw)]
        for cp in pf_cps:
            cp.start()

        def edge_copies(c, b):
            base = ehalf * epw + c * ec
            return (
                pltpu.make_async_copy(
                    pk_hbm.at[pl.ds(base, ec)], pk_b[b], sems.at[2 * b]),
                pltpu.make_async_copy(
                    nw_hbm.at[pl.ds(base, ec)], nw_b[b], sems.at[2 * b + 1]),
            )

        def issue(c, b):
            for cp in edge_copies(c, b):
                cp.start()

        issue(0, 0)
        issue(1, 1)

        @plsc.parallel_loop(0, N // L, unroll=8)
        def zero(i):
            z = jnp.zeros((L,), jnp.float32)
            for j in range(fpw):
                gf[j][pl.ds(i * L, L)] = z

        for cp in pf_cps:
            cp.wait()

        def process(c, b):
            for cp in edge_copies(c, b):
                cp.wait()

            @plsc.parallel_loop(0, ec // L, unroll=8)
            def grp(g):
                pk = pk_b[b][pl.ds(g * L, L)]
                w = nw_b[b][pl.ds(g * L, L)]
                s = pk & 0xFFFF
                d = lax.shift_right_logical(pk, 16)
                for j in range(fpw):
                    v = plsc.load_gather(pf[j], [s])
                    plsc.addupdate_scatter(gf[j], [d], v * w)

        def chunk2(k, carry):
            c0 = 2 * k
            process(c0, 0)

            @pl.when(c0 + 2 < nchunks)
            def _():
                issue(c0 + 2, 0)
            process(c0 + 1, 1)

            @pl.when(c0 + 3 < nchunks)
            def _():
                issue(c0 + 3, 1)
            return carry
        lax.fori_loop(0, nchunks // 2, chunk2, 0)

        gf_cps = [pltpu.make_async_copy(gf[j], gt_out.at[ehalf, f0 + j],
                                        psems.at[j])
                  for j in range(fpw)]
        for cp in gf_cps:
            cp.start()
        for cp in gf_cps:
            cp.wait()

    scratch = ([pltpu.VMEM((N,), jnp.float32)] * (2 * fpw)
               + [pltpu.VMEM((ec,), jnp.int32)] * 2
               + [pltpu.VMEM((ec,), jnp.float32)] * 2
               + [pltpu.SemaphoreType.DMA((4,)),
                  pltpu.SemaphoreType.DMA((fpw,))])
    return pl.kernel(
        body,
        out_type=jax.ShapeDtypeStruct((2, F, N), jnp.float32),
        mesh=_mesh(),
        scratch_types=scratch,
        compiler_params=pltpu.CompilerParams(needs_layout_passes=False),
    )


def _deg_kernel():
    return pl.kernel(
        _deg_body,
        out_type=jax.ShapeDtypeStruct((NWORK, N), jnp.float32),
        mesh=_mesh(),
        scratch_types=[pltpu.VMEM((N,), jnp.float32),
                       pltpu.VMEM((EPW,), jnp.int32),
                       pltpu.VMEM((EPW,), jnp.float32),
                       pltpu.SemaphoreType.DMA((2,))],
        compiler_params=pltpu.CompilerParams(needs_layout_passes=False),
    )


def _nw_kernel():
    return pl.kernel(
        _nw_body,
        out_type=[jax.ShapeDtypeStruct((E,), jnp.float32),
                  jax.ShapeDtypeStruct((E,), jnp.int32)],
        mesh=_mesh(),
        scratch_types=[pltpu.VMEM((N,), jnp.float32),
                       pltpu.VMEM((EPW,), jnp.int32),
                       pltpu.VMEM((EPW,), jnp.int32),
                       pltpu.VMEM((EPW,), jnp.float32),
                       pltpu.VMEM((EPW,), jnp.float32),
                       pltpu.VMEM((EPW,), jnp.int32),
                       pltpu.SemaphoreType.DMA((4,))],
        compiler_params=pltpu.CompilerParams(needs_layout_passes=False),
    )


def _tc1_body(xt_ref, degp_ref, wd_ref, wp_ref, d1_ref, p1_ref, dinv_ref):
    deg = jnp.sum(degp_ref[...], axis=0, keepdims=True)
    deg_safe = jnp.where(deg > 0, deg, 1.0)
    dinv_ref[...] = jnp.where(deg > 0, lax.rsqrt(deg_safe), 0.0)
    xt = xt_ref[...]
    d1_ref[...] = jnp.dot(wd_ref[...], xt, preferred_element_type=jnp.float32)
    p1_ref[...] = jnp.dot(wp_ref[...], xt, preferred_element_type=jnp.float32)


def _tc1_call(xt, degp, wdt, wpt):
    return pl.pallas_call(
        _tc1_body,
        out_shape=[
            jax.ShapeDtypeStruct((64, N), jnp.float32),
            jax.ShapeDtypeStruct((64, N), jnp.float32),
            jax.ShapeDtypeStruct((1, N), jnp.float32),
        ],
    )(xt, degp, wdt, wpt)


def _tc2_body(d1_ref, g1_ref, bc_ref, wd_ref, wp_ref, d2_ref, p2_ref):
    pre = d1_ref[...] - (g1_ref[0] + g1_ref[1]) + bc_ref[...]
    z = jax.nn.sigmoid(pre[:32, :])
    ht = jnp.tanh(pre[32:, :])
    h = jax.nn.relu((1.0 - z) * ht)
    d2_ref[...] = jnp.dot(wd_ref[...], h, preferred_element_type=jnp.float32)
    p2_ref[...] = jnp.dot(wp_ref[...], h, preferred_element_type=jnp.float32)


def _tc2_call(d1t, g1t, bct, wdt, wpt):
    return pl.pallas_call(
        _tc2_body,
        out_shape=[
            jax.ShapeDtypeStruct((32, N), jnp.float32),
            jax.ShapeDtypeStruct((32, N), jnp.float32),
        ],
    )(d1t, g1t, bct, wdt, wpt)


def _tc3_body(d2_ref, g2_ref, bc_ref, lw_ref, lb_ref, out_ref):
    pre = d2_ref[...] - (g2_ref[0] + g2_ref[1]) + bc_ref[...]
    z = jax.nn.sigmoid(pre[:16, :])
    ht = jnp.tanh(pre[16:, :])
    h = jax.nn.relu((1.0 - z) * ht)
    logits = jnp.dot(lw_ref[...], h, preferred_element_type=jnp.float32)
    logits = logits + lb_ref[...]
    m = jnp.max(logits, axis=0, keepdims=True)
    shifted = logits - m
    ssum = jnp.sum(jnp.exp(shifted), axis=0, keepdims=True)
    out_ref[...] = shifted - jnp.log(ssum)


def _tc3_call(d2t, g2t, bct, lwt, lbt):
    return pl.pallas_call(
        _tc3_body,
        out_shape=jax.ShapeDtypeStruct((10, N), jnp.float32),
    )(d2t, g2t, bct, lwt, lbt)


def kernel(x, edge_index, edge_weight, params):
    src = edge_index[0]
    dst = edge_index[1]
    g1, g2 = params["gru1"], params["gru2"]
    wd1t = jnp.concatenate([g1["xz"]["W"][0], g1["xh"]["W"][0]], axis=1).T
    wp1t = jnp.concatenate([g1["xz"]["W"][1], g1["xh"]["W"][1]], axis=1).T
    bc1t = jnp.concatenate([g1["xz"]["b"] + g1["hz"]["b"],
                            g1["xh"]["b"] + g1["hh"]["b"]])[:, None]
    wd2t = jnp.concatenate([g2["xz"]["W"][0], g2["xh"]["W"][0]], axis=1).T
    wp2t = jnp.concatenate([g2["xz"]["W"][1], g2["xh"]["W"][1]], axis=1).T
    bc2t = jnp.concatenate([g2["xz"]["b"] + g2["hz"]["b"],
                            g2["xh"]["b"] + g2["hh"]["b"]])[:, None]
    lwt = params["lin_W"].T
    lbt = params["lin_b"][:, None]

    degp = _deg_kernel()(edge_weight, src)
    d1t, p1t, dinv2d = _tc1_call(x.T, degp, wd1t, wp1t)
    dinv = dinv2d.reshape(N)
    nw, pk = _nw_kernel()(dinv, src, dst, edge_weight)
    g1t = _make_prop(64, 4)(p1t, pk, nw)
    d2t, p2t = _tc2_call(d1t, g1t, bc1t, wd2t, wp2t)
    g2t = _make_prop(32, 2)(p2t, pk, nw)
    return _tc3_call(d2t, g2t, bc2t, lwt, lbt).T


# speedup vs baseline: 1.0445x; 1.0047x over previous
import functools

import jax
import jax.numpy as jnp
from jax import lax
from jax.experimental import pallas as pl
from jax.experimental.pallas import tpu as pltpu
from jax.experimental.pallas import tpu_sc as plsc

N = 10000
E = 320000
L = 16
NCORES = 2
NSUB = 16
NWORK = NCORES * NSUB
EPW = E // NWORK
EC = 8000

NB = 2000


def _mesh():
    return plsc.VectorSubcoreMesh(
        core_axis_name="c", subcore_axis_name="s",
        num_cores=NCORES, num_subcores=NSUB)


def _wid():
    return lax.axis_index("s") * NCORES + lax.axis_index("c")


def _deg_body(ew_hbm, src_hbm, degp_out, acc_v, src_v, ew_v, sems):
    wid = _wid()
    base = wid * EPW
    cps = (pltpu.make_async_copy(src_hbm.at[pl.ds(base, EPW)], src_v,
                                 sems.at[0]),
           pltpu.make_async_copy(ew_hbm.at[pl.ds(base, EPW)], ew_v,
                                 sems.at[1]))
    for cp in cps:
        cp.start()

    @plsc.parallel_loop(0, N // L, unroll=8)
    def zero(i):
        acc_v[pl.ds(i * L, L)] = jnp.zeros((L,), jnp.float32)
    for cp in cps:
        cp.wait()

    @plsc.parallel_loop(0, EPW // L, unroll=8)
    def body(g):
        s = src_v[pl.ds(g * L, L)]
        w = ew_v[pl.ds(g * L, L)]
        plsc.addupdate_scatter(acc_v, [s], w)
    pltpu.sync_copy(acc_v, degp_out.at[wid])


def _nw_body(dinv_hbm, src_hbm, dst_hbm, ew_hbm, nw_out, pk_out,
             dinv_v, src_v, dst_v, ew_v, nw_v, pk_v, sems):
    wid = _wid()
    base = wid * EPW
    cps = (pltpu.make_async_copy(dinv_hbm, dinv_v, sems.at[0]),
           pltpu.make_async_copy(src_hbm.at[pl.ds(base, EPW)], src_v,
                                 sems.at[1]),
           pltpu.make_async_copy(dst_hbm.at[pl.ds(base, EPW)], dst_v,
                                 sems.at[2]),
           pltpu.make_async_copy(ew_hbm.at[pl.ds(base, EPW)], ew_v,
                                 sems.at[3]))
    for cp in cps:
        cp.start()
    for cp in cps:
        cp.wait()

    @plsc.parallel_loop(0, EPW // L, unroll=8)
    def body(g):
        s = src_v[pl.ds(g * L, L)]
        d = dst_v[pl.ds(g * L, L)]
        w = ew_v[pl.ds(g * L, L)]
        a = plsc.load_gather(dinv_v, [s])
        b = plsc.load_gather(dinv_v, [d])
        nw_v[pl.ds(g * L, L)] = a * w * b
        pk_v[pl.ds(g * L, L)] = s | (d << 16)
    pltpu.sync_copy(nw_v, nw_out.at[pl.ds(base, EPW)])
    pltpu.sync_copy(pk_v, pk_out.at[pl.ds(base, EPW)])


def _make_prop(F, fpw):
    assert F == fpw * NSUB
    epw = E // 2
    ec = EC
    nchunks = epw // ec
    assert nchunks % 2 == 0 and epw % ec == 0

    def body(pt_hbm, pk_hbm, nw_hbm, gt_out, *scratch):
        pf = scratch[:fpw]
        gf = scratch[fpw:2 * fpw]
        pk_b = scratch[2 * fpw:2 * fpw + 2]
        nw_b = scratch[2 * fpw + 2:2 * fpw + 4]
        sems = scratch[2 * fpw + 4]
        psems = scratch[2 * fpw + 5]
        ehalf = lax.axis_index("c")
        f0 = lax.axis_index("s") * fpw
        pf_cps = [pltpu.make_async_copy(pt_hbm.at[f0 + j], pf[j],
                                        psems.at[j])
                  for j in range(fpw)]
        for cp in pf_cps:
            cp.start()

        def edge_copies(c, b):
            base = ehalf * epw + c * ec
            return (
                pltpu.make_async_copy(
                    pk_hbm.at[pl.ds(base, ec)], pk_b[b], sems.at[2 * b]),
                pltpu.make_async_copy(
                    nw_hbm.at[pl.ds(base, ec)], nw_b[b], sems.at[2 * b + 1]),
            )

        def issue(c, b):
            for cp in edge_copies(c, b):
                cp.start()

        issue(0, 0)
        issue(1, 1)

        @plsc.parallel_loop(0, N // L, unroll=8)
        def zero(i):
            z = jnp.zeros((L,), jnp.float32)
            for j in range(fpw):
                gf[j][pl.ds(i * L, L)] = z

        for cp in pf_cps:
            cp.wait()

        def process(c, b):
            for cp in edge_copies(c, b):
                cp.wait()

            @plsc.parallel_loop(0, ec // L, unroll=8)
            def grp(g):
                pk = pk_b[b][pl.ds(g * L, L)]
                w = nw_b[b][pl.ds(g * L, L)]
                s = pk & 0xFFFF
                d = lax.shift_right_logical(pk, 16)
                for j in range(fpw):
                    v = plsc.load_gather(pf[j], [s])
                    plsc.addupdate_scatter(gf[j], [d], v * w)

        def chunk2(k, carry):
            c0 = 2 * k
            process(c0, 0)

            @pl.when(c0 + 2 < nchunks)
            def _():
                issue(c0 + 2, 0)
            process(c0 + 1, 1)

            @pl.when(c0 + 3 < nchunks)
            def _():
                issue(c0 + 3, 1)
            return carry
        lax.fori_loop(0, nchunks // 2, chunk2, 0)

        gf_cps = [pltpu.make_async_copy(gf[j], gt_out.at[ehalf, f0 + j],
                                        psems.at[j])
                  for j in range(fpw)]
        for cp in gf_cps:
            cp.start()
        for cp in gf_cps:
            cp.wait()

    scratch = ([pltpu.VMEM((N,), jnp.float32)] * (2 * fpw)
               + [pltpu.VMEM((ec,), jnp.int32)] * 2
               + [pltpu.VMEM((ec,), jnp.float32)] * 2
               + [pltpu.SemaphoreType.DMA((4,)),
                  pltpu.SemaphoreType.DMA((fpw,))])
    return pl.kernel(
        body,
        out_type=jax.ShapeDtypeStruct((2, F, N), jnp.float32),
        mesh=_mesh(),
        scratch_types=scratch,
        compiler_params=pltpu.CompilerParams(needs_layout_passes=False),
    )


def _deg_kernel():
    return pl.kernel(
        _deg_body,
        out_type=jax.ShapeDtypeStruct((NWORK, N), jnp.float32),
        mesh=_mesh(),
        scratch_types=[pltpu.VMEM((N,), jnp.float32),
                       pltpu.VMEM((EPW,), jnp.int32),
                       pltpu.VMEM((EPW,), jnp.float32),
                       pltpu.SemaphoreType.DMA((2,))],
        compiler_params=pltpu.CompilerParams(needs_layout_passes=False),
    )


def _nw_kernel():
    return pl.kernel(
        _nw_body,
        out_type=[jax.ShapeDtypeStruct((E,), jnp.float32),
                  jax.ShapeDtypeStruct((E,), jnp.int32)],
        mesh=_mesh(),
        scratch_types=[pltpu.VMEM((N,), jnp.float32),
                       pltpu.VMEM((EPW,), jnp.int32),
                       pltpu.VMEM((EPW,), jnp.int32),
                       pltpu.VMEM((EPW,), jnp.float32),
                       pltpu.VMEM((EPW,), jnp.float32),
                       pltpu.VMEM((EPW,), jnp.int32),
                       pltpu.SemaphoreType.DMA((4,))],
        compiler_params=pltpu.CompilerParams(needs_layout_passes=False),
    )


def _tc1_body(x_ref, degp_ref, wd_ref, wp_ref, d1_ref, p1_ref, dinv_ref):
    deg = jnp.sum(degp_ref[...], axis=0, keepdims=True)
    deg_safe = jnp.where(deg > 0, deg, 1.0)
    dinv_ref[...] = jnp.where(deg > 0, lax.rsqrt(deg_safe), 0.0)
    xb = x_ref[...]
    dn = (((1,), (1,)), ((), ()))
    d1_ref[...] = lax.dot_general(wd_ref[...], xb, dimension_numbers=dn,
                                  preferred_element_type=jnp.float32)
    p1_ref[...] = lax.dot_general(wp_ref[...], xb, dimension_numbers=dn,
                                  preferred_element_type=jnp.float32)


def _tc1_call(x, degp, wdt, wpt):
    return pl.pallas_call(
        _tc1_body,
        out_shape=[
            jax.ShapeDtypeStruct((64, N), jnp.float32),
            jax.ShapeDtypeStruct((64, N), jnp.float32),
            jax.ShapeDtypeStruct((1, N), jnp.float32),
        ],
    )(x, degp, wdt, wpt)


def _tc2_body(d1_ref, g1_ref, bc_ref, wd_ref, wp_ref, d2_ref, p2_ref):
    pre = d1_ref[...] - (g1_ref[0] + g1_ref[1]) + bc_ref[...]
    z = jax.nn.sigmoid(pre[:32, :])
    ht = jnp.tanh(pre[32:, :])
    h = jax.nn.relu((1.0 - z) * ht)
    d2_ref[...] = jnp.dot(wd_ref[...], h, preferred_element_type=jnp.float32)
    p2_ref[...] = jnp.dot(wp_ref[...], h, preferred_element_type=jnp.float32)


def _tc2_call(d1t, g1t, bct, wdt, wpt):
    return pl.pallas_call(
        _tc2_body,
        out_shape=[
            jax.ShapeDtypeStruct((32, N), jnp.float32),
            jax.ShapeDtypeStruct((32, N), jnp.float32),
        ],
    )(d1t, g1t, bct, wdt, wpt)


def _tc3_body(d2_ref, g2_ref, bc_ref, lw_ref, lb_ref, out_ref):
    pre = d2_ref[...] - (g2_ref[0] + g2_ref[1]) + bc_ref[...]
    z = jax.nn.sigmoid(pre[:16, :])
    ht = jnp.tanh(pre[16:, :])
    h = jax.nn.relu((1.0 - z) * ht)
    logits = jnp.dot(lw_ref[...], h, preferred_element_type=jnp.float32)
    logits = logits + lb_ref[...]
    m = jnp.max(logits, axis=0, keepdims=True)
    shifted = logits - m
    ssum = jnp.sum(jnp.exp(shifted), axis=0, keepdims=True)
    out_ref[...] = shifted - jnp.log(ssum)


def _tc3_call(d2t, g2t, bct, lwt, lbt):
    return pl.pallas_call(
        _tc3_body,
        out_shape=jax.ShapeDtypeStruct((10, N), jnp.float32),
    )(d2t, g2t, bct, lwt, lbt)


def kernel(x, edge_index, edge_weight, params):
    src = edge_index[0]
    dst = edge_index[1]
    g1, g2 = params["gru1"], params["gru2"]
    wd1t = jnp.concatenate([g1["xz"]["W"][0], g1["xh"]["W"][0]], axis=1).T
    wp1t = jnp.concatenate([g1["xz"]["W"][1], g1["xh"]["W"][1]], axis=1).T
    bc1t = jnp.concatenate([g1["xz"]["b"] + g1["hz"]["b"],
                            g1["xh"]["b"] + g1["hh"]["b"]])[:, None]
    wd2t = jnp.concatenate([g2["xz"]["W"][0], g2["xh"]["W"][0]], axis=1).T
    wp2t = jnp.concatenate([g2["xz"]["W"][1], g2["xh"]["W"][1]], axis=1).T
    bc2t = jnp.concatenate([g2["xz"]["b"] + g2["hz"]["b"],
                            g2["xh"]["b"] + g2["hh"]["b"]])[:, None]
    lwt = params["lin_W"].T
    lbt = params["lin_b"][:, None]

    degp = _deg_kernel()(edge_weight, src)
    d1t, p1t, dinv2d = _tc1_call(x, degp, wd1t, wp1t)
    dinv = dinv2d.reshape(N)
    nw, pk = _nw_kernel()(dinv, src, dst, edge_weight)
    g1t = _make_prop(64, 4)(p1t, pk, nw)
    d2t, p2t = _tc2_call(d1t, g1t, bc1t, wd2t, wp2t)
    g2t = _make_prop(32, 2)(p2t, pk, nw)
    return _tc3_call(d2t, g2t, bc2t, lwt, lbt).T
